# Initial kernel scaffold; baseline (speedup 1.0000x reference)
import jax
import jax.numpy as jnp
from jax.experimental import pallas as pl


def kernel(values, lengths, position_weights):
    raise NotImplementedError("write your pallas kernel here")


# speedup vs baseline: 749.9229x; 749.9229x over previous
import functools

import jax
import jax.numpy as jnp
from jax import lax
from jax.experimental import pallas as pl
from jax.experimental.pallas import tpu as pltpu
from jax.experimental.pallas import tpu_sc as plsc

N_KEYS = 26
BATCH = 1024
MAX_LEN = 200
NBAGS = N_KEYS * BATCH
NC = 2
NS = 16
L = 16
NW = NC * NS
BAGS_PER_SUB = NBAGS // NS
VEC_PER_SUB = BAGS_PER_SUB // L
PW_FLAT = N_KEYS * MAX_LEN
PW_PAD = 5216
OFF_PAD = NBAGS + 2 * L
NVEC_BAG = (MAX_LEN + L - 1) // L
INT_MAX = 2**31 - 1
CAP = 89600


def _sload(ref, i):
    return ref[pl.ds(i, L)][0]


def _pwp_body(T, C, nchunks, len_hbm, pw_hbm, out_hbm,
              len_v, tot_v, tot_all_v, offs_v, pw_v, buf_v,
              tot_sh, off_sh):
    s = lax.axis_index("s")
    c = lax.axis_index("c")
    wid = s * NC + c
    lane = jnp.arange(L, dtype=jnp.int32)

    base_bag = s * BAGS_PER_SUB
    pltpu.sync_copy(len_hbm.at[pl.ds(base_bag, BAGS_PER_SUB)], len_v)

    def scan_body(k, carry):
        vec = len_v[pl.ds(k * L, L)]
        inc = plsc.cumsum(vec)
        len_v[pl.ds(k * L, L)] = inc - vec + carry
        return carry + jnp.sum(vec)

    carry = lax.fori_loop(0, VEC_PER_SUB, scan_body, jnp.int32(0))

    tot_v[...] = jnp.full((L,), carry, jnp.int32)
    pltpu.sync_copy(tot_v.at[pl.ds(0, 8)], tot_sh.at[pl.ds(s * 8, 8)])
    plsc.subcore_barrier()
    pltpu.sync_copy(tot_sh, tot_all_v)
    tots = plsc.load_gather(tot_all_v, [lane * 8])
    base = jnp.sum(jnp.where(lane < s, tots, 0))
    total_sum = jnp.sum(tots)

    def add_body(k, unused):
        len_v[pl.ds(k * L, L)] = len_v[pl.ds(k * L, L)] + base
        return unused

    lax.fori_loop(0, VEC_PER_SUB, add_body, jnp.int32(0))
    pltpu.sync_copy(len_v, off_sh.at[pl.ds(base_bag, BAGS_PER_SUB)])

    @pl.when(s == NS - 1)
    def _():
        tot_v[...] = jnp.where(lane == 0, total_sum, jnp.int32(INT_MAX))
        pltpu.sync_copy(tot_v, off_sh.at[pl.ds(NBAGS, L)])

    plsc.subcore_barrier()
    pltpu.sync_copy(off_sh, offs_v)

    pltpu.sync_copy(pw_hbm, pw_v)
    p0w = wid * T

    def bs_body(unused, lohi):
        lo, hi = lohi
        mid = (lo + hi + 1) // 2
        le = _sload(offs_v, mid) <= p0w
        return (jnp.where(le, mid, lo), jnp.where(le, hi, mid - 1))

    b, _ = lax.fori_loop(0, 15, bs_body,
                         (jnp.int32(0), jnp.int32(NBAGS)))
    ob = _sload(offs_v, b)

    for ci in range(nchunks):
        p0 = p0w + ci * C
        csz = min(C, T - ci * C)
        p1 = p0 + csz

        def walk_body(st):
            b, ob, _ = st
            oe = _sload(offs_v, b + 1)
            key = jnp.minimum(b >> 10, N_KEYS - 1)
            src0 = key * MAX_LEN
            jj0 = jnp.maximum((p0 - ob) >> 4, 0)
            dst0 = ob - p0 + L
            for j in range(NVEC_BAG):
                vec = pw_v[pl.ds(src0 + j * L, L)]
                d = jnp.where(j >= jj0, dst0 + j * L, 0)
                buf_v[pl.ds(d, L)] = vec
            cont = oe < p1
            return (jnp.where(cont, b + 1, b), jnp.where(cont, oe, ob), cont)

        b, ob, _ = lax.while_loop(lambda st: st[2], walk_body,
                                  (b, ob, ob < p1))
        pltpu.sync_copy(buf_v.at[pl.ds(L, csz)],
                        out_hbm.at[pl.ds(p0, csz)])


def kernel(values, lengths, position_weights):
    total = values.shape[0]
    if total == 0:
        return values, jnp.zeros((0,), jnp.float32), lengths

    per_w = (total + NW - 1) // NW
    T = (per_w + L - 1) // L * L
    nchunks = (T + CAP - 1) // CAP
    C = min(T, CAP)

    pw_flat = jnp.concatenate([
        position_weights.reshape(-1),
        jnp.zeros((PW_PAD - PW_FLAT,), jnp.float32),
    ])

    mesh = plsc.VectorSubcoreMesh(core_axis_name="c", subcore_axis_name="s")
    out = pl.kernel(
        functools.partial(_pwp_body, T, C, nchunks),
        out_type=jax.ShapeDtypeStruct((NW * T,), jnp.float32),
        mesh=mesh,
        compiler_params=pltpu.CompilerParams(needs_layout_passes=False),
        scratch_types=[
            pltpu.VMEM((BAGS_PER_SUB,), jnp.int32),
            pltpu.VMEM((L,), jnp.int32),
            pltpu.VMEM((NS * 8,), jnp.int32),
            pltpu.VMEM((OFF_PAD,), jnp.int32),
            pltpu.VMEM((PW_PAD,), jnp.float32),
            pltpu.VMEM((C + 256,), jnp.float32),
            pltpu.VMEM_SHARED((NS * 8,), jnp.int32),
            pltpu.VMEM_SHARED((OFF_PAD,), jnp.int32),
        ],
    )(lengths, pw_flat)
    return values, out[:total], lengths
